# baseline (speedup 1.0000x reference)
import functools

import jax
import jax.numpy as jnp
from jax import lax
from jax.experimental import pallas as pl
from jax.experimental.pallas import tpu as pltpu
from jax.experimental.pallas import tpu_sc as plsc

DIM = 300
SEQ = 200
TOK = 180
BATCH = 1024
TOK_PAD = 192
PAD = SEQ - TOK
GROUPS = TOK // 16
REM = TOK - GROUPS * 16


@functools.lru_cache(maxsize=1)
def _make_sc_gather():
    info = plsc.get_sparse_core_info()
    nw = info.num_cores * info.num_subcores
    bpw = BATCH // nw
    mesh = plsc.VectorSubcoreMesh(core_axis_name="c", subcore_axis_name="s")

    @functools.partial(
        pl.kernel,
        mesh=mesh,
        compiler_params=pltpu.CompilerParams(use_tc_tiling_on_sc=False),
        out_type=jax.ShapeDtypeStruct((BATCH, SEQ, DIM), jnp.float32),
        scratch_types=[
            pltpu.VMEM((bpw, TOK_PAD), jnp.int32),
            pltpu.VMEM((TOK, DIM), jnp.float32),
            pltpu.VMEM((TOK, DIM), jnp.float32),
            pltpu.VMEM((PAD, DIM), jnp.float32),
            pltpu.SemaphoreType.DMA,
            pltpu.SemaphoreType.DMA,
            pltpu.SemaphoreType.DMA,
            pltpu.SemaphoreType.DMA,
        ],
    )
    def gather_kernel(idx_hbm, zeros_hbm, table_hbm, out_hbm,
                      idx_v, buf0, buf1, zbuf, g0, g1, s0, s1):
        wid = lax.axis_index("s") * info.num_cores + lax.axis_index("c")
        b0 = wid * bpw

        pltpu.sync_copy(idx_hbm.at[pl.ds(b0, bpw)], idx_v)
        pltpu.sync_copy(zeros_hbm, zbuf)

        def fetch_sentence(j, buf, gsem):
            descs = []
            for g in range(GROUPS + 1):
                nlanes = 16 if g < GROUPS else REM
                v = idx_v[j, pl.ds(16 * g, 16)]
                for t in range(nlanes):
                    descs.append(pltpu.async_copy(
                        table_hbm.at[v[t]], buf.at[16 * g + t], gsem))
            return descs

        def store_sentence(j, buf, ssem, zsem):
            a = pltpu.async_copy(buf, out_hbm.at[b0 + j, pl.ds(0, TOK)], ssem)
            z = pltpu.async_copy(zbuf, out_hbm.at[b0 + j, pl.ds(TOK, PAD)], zsem)
            return a, z

        def body(i, carry):
            j0 = 2 * i
            j1 = 2 * i + 1
            ga = fetch_sentence(j0, buf0, g0)
            gb = fetch_sentence(j1, buf1, g1)
            for d in ga:
                d.wait()
            sa, za = store_sentence(j0, buf0, s0, s1)
            for d in gb:
                d.wait()
            sb, zb = store_sentence(j1, buf1, s0, s1)
            sa.wait()
            za.wait()
            sb.wait()
            zb.wait()
            return carry

        lax.fori_loop(0, bpw // 2, body, 0)

    return gather_kernel


def kernel(indices, table):
    idx = jnp.pad(indices, ((0, 0), (0, TOK_PAD - TOK)))
    zeros = jnp.zeros((PAD, DIM), jnp.float32)
    return _make_sc_gather()(idx, zeros, table)
